# baseline (speedup 1.0000x reference)
import functools

import jax
import jax.numpy as jnp
from jax import lax
from jax.experimental import pallas as pl
from jax.experimental.pallas import tpu as pltpu
from jax.experimental.pallas import tpu_sc as plsc

BATCH = 4
SEQ = 8192
HID = 2048
LANES = 16
UNROLL = 16
CHUNKS = SEQ // LANES
OUTER = CHUNKS // UNROLL


def _sc_body(hid_hbm, pids_hbm, out_hbm, pids_v, row_v):
    nc = 2
    wid = lax.axis_index("s") * nc + lax.axis_index("c")

    @pl.when(wid < BATCH)
    def _():
        b = wid
        pltpu.sync_copy(pids_hbm.at[b], pids_v)

        init_max = jnp.full((LANES,), jnp.int32(-2147483648), jnp.int32)
        init_chunk = jnp.zeros((LANES,), jnp.int32)

        def body(i, carry):
            cur_max, cur_chunk = carry
            for u in range(UNROLL):
                c = i * UNROLL + u
                v = pids_v[pl.ds(c * LANES, LANES)]
                take = v > cur_max
                cur_max = jnp.where(take, v, cur_max)
                cur_chunk = jnp.where(take, c, cur_chunk)
            return (cur_max, cur_chunk)

        cur_max, cur_chunk = lax.fori_loop(
            0, OUTER, body, (init_max, init_chunk)
        )
        cur_idx = cur_chunk * LANES + lax.iota(jnp.int32, LANES)

        best_val = cur_max[0]
        best_idx = cur_idx[0]
        for j in range(1, LANES):
            v = cur_max[j]
            i = cur_idx[j]
            take = (v > best_val) | ((v == best_val) & (i < best_idx))
            best_val = jnp.where(take, v, best_val)
            best_idx = jnp.where(take, i, best_idx)
        idx = best_idx

        pltpu.sync_copy(hid_hbm.at[b, pl.ds(idx, 1)], out_hbm.at[b])


@jax.jit
def _sc_kernel(hidden_states, position_ids):
    return pl.kernel(
        _sc_body,
        mesh=plsc.VectorSubcoreMesh(core_axis_name="c", subcore_axis_name="s"),
        out_type=jax.ShapeDtypeStruct((BATCH, 1, HID), jnp.float32),
        scratch_types=[
            pltpu.VMEM((SEQ,), jnp.int32),
            pltpu.VMEM((1, HID), jnp.float32),
        ],
    )(hidden_states, position_ids)


def kernel(hidden_states, position_ids):
    return _sc_kernel(hidden_states, position_ids)


# speedup vs baseline: 1.0096x; 1.0096x over previous
import functools

import jax
import jax.numpy as jnp
from jax import lax
from jax.experimental import pallas as pl
from jax.experimental.pallas import tpu as pltpu
from jax.experimental.pallas import tpu_sc as plsc

BATCH = 4
SEQ = 8192
HID = 2048
LANES = 16
UNROLL = 8
CHUNKS = SEQ // LANES
OUTER = CHUNKS // UNROLL


def _sc_body(hid_hbm, pids_hbm, out_hbm, pids_v, row_v):
    nc = 2
    wid = lax.axis_index("s") * nc + lax.axis_index("c")

    @pl.when(wid < BATCH)
    def _():
        b = wid
        pltpu.sync_copy(pids_hbm.at[b], pids_v)

        init_max = jnp.full((LANES,), jnp.int32(-2147483648), jnp.int32)
        init_chunk = jnp.zeros((LANES,), jnp.int32)

        def body(i, carry):
            cur_max, cur_chunk = carry
            for u in range(UNROLL):
                c = i * UNROLL + u
                v = pids_v[pl.ds(c * LANES, LANES)]
                take = v > cur_max
                cur_max = jnp.where(take, v, cur_max)
                cur_chunk = jnp.where(take, c, cur_chunk)
            return (cur_max, cur_chunk)

        cur_max, cur_chunk = lax.fori_loop(
            0, OUTER, body, (init_max, init_chunk)
        )
        cur_idx = cur_chunk * LANES + lax.iota(jnp.int32, LANES)

        best_val = cur_max[0]
        best_idx = cur_idx[0]
        for j in range(1, LANES):
            v = cur_max[j]
            i = cur_idx[j]
            take = (v > best_val) | ((v == best_val) & (i < best_idx))
            best_val = jnp.where(take, v, best_val)
            best_idx = jnp.where(take, i, best_idx)
        idx = best_idx

        pltpu.sync_copy(hid_hbm.at[b, pl.ds(idx, 1)], out_hbm.at[b])


@jax.jit
def _sc_kernel(hidden_states, position_ids):
    return pl.kernel(
        _sc_body,
        mesh=plsc.VectorSubcoreMesh(core_axis_name="c", subcore_axis_name="s"),
        out_type=jax.ShapeDtypeStruct((BATCH, 1, HID), jnp.float32),
        scratch_types=[
            pltpu.VMEM((SEQ,), jnp.int32),
            pltpu.VMEM((1, HID), jnp.float32),
        ],
    )(hidden_states, position_ids)


def kernel(hidden_states, position_ids):
    return _sc_kernel(hidden_states, position_ids)


# speedup vs baseline: 1.0278x; 1.0180x over previous
import functools

import jax
import jax.numpy as jnp
from jax import lax
from jax.experimental import pallas as pl
from jax.experimental.pallas import tpu as pltpu
from jax.experimental.pallas import tpu_sc as plsc

BATCH = 4
SEQ = 8192
HID = 2048
LANES = 16
UNROLL = 8
CHUNKS = SEQ // LANES
HALF_CHUNKS = CHUNKS // 2
HALF_SEQ = SEQ // 2
HALF_HID = HID // 2


def _sc_body(hid_hbm, pids_hbm, out_hbm, pids_v, row_v, sem0, sem1):
    nc = 2
    wid = lax.axis_index("s") * nc + lax.axis_index("c")

    @pl.when(wid < 2 * BATCH)
    def _():
        b = wid % BATCH
        half = wid // BATCH

        cp0 = pltpu.async_copy(
            pids_hbm.at[b, pl.ds(0, HALF_SEQ)], pids_v.at[pl.ds(0, HALF_SEQ)],
            sem0,
        )
        cp1 = pltpu.async_copy(
            pids_hbm.at[b, pl.ds(HALF_SEQ, HALF_SEQ)],
            pids_v.at[pl.ds(HALF_SEQ, HALF_SEQ)],
            sem1,
        )

        init_max = jnp.full((LANES,), jnp.int32(-2147483648), jnp.int32)
        init_chunk = jnp.zeros((LANES,), jnp.int32)

        def body(i, carry):
            cur_max, cur_chunk = carry
            for u in range(UNROLL):
                c = i * UNROLL + u
                v = pids_v[pl.ds(c * LANES, LANES)]
                take = v > cur_max
                cur_max = jnp.where(take, v, cur_max)
                cur_chunk = jnp.where(take, c, cur_chunk)
            return (cur_max, cur_chunk)

        cp0.wait()
        carry = lax.fori_loop(
            0, HALF_CHUNKS // UNROLL, body, (init_max, init_chunk)
        )
        cp1.wait()
        cur_max, cur_chunk = lax.fori_loop(
            HALF_CHUNKS // UNROLL, CHUNKS // UNROLL, body, carry
        )
        cur_idx = cur_chunk * LANES + lax.iota(jnp.int32, LANES)

        best_val = cur_max[0]
        best_idx = cur_idx[0]
        for j in range(1, LANES):
            v = cur_max[j]
            i = cur_idx[j]
            take = (v > best_val) | ((v == best_val) & (i < best_idx))
            best_val = jnp.where(take, v, best_val)
            best_idx = jnp.where(take, i, best_idx)
        idx = best_idx

        h0 = half * HALF_HID
        pltpu.sync_copy(hid_hbm.at[b, pl.ds(idx, 1), pl.ds(h0, HALF_HID)],
                        row_v)
        pltpu.sync_copy(row_v, out_hbm.at[b, pl.ds(0, 1), pl.ds(h0, HALF_HID)])


@jax.jit
def _sc_kernel(hidden_states, position_ids):
    return pl.kernel(
        _sc_body,
        mesh=plsc.VectorSubcoreMesh(core_axis_name="c", subcore_axis_name="s"),
        out_type=jax.ShapeDtypeStruct((BATCH, 1, HID), jnp.float32),
        scratch_types=[
            pltpu.VMEM((SEQ,), jnp.int32),
            pltpu.VMEM((1, HALF_HID), jnp.float32),
            pltpu.SemaphoreType.DMA,
            pltpu.SemaphoreType.DMA,
        ],
    )(hidden_states, position_ids)


def kernel(hidden_states, position_ids):
    return _sc_kernel(hidden_states, position_ids)


# speedup vs baseline: 1.0534x; 1.0249x over previous
import functools

import jax
import jax.numpy as jnp
from jax import lax
from jax.experimental import pallas as pl
from jax.experimental.pallas import tpu as pltpu
from jax.experimental.pallas import tpu_sc as plsc

BATCH = 4
SEQ = 8192
HID = 2048
LANES = 16
UNROLL = 8
CHUNKS = SEQ // LANES
HALF_CHUNKS = CHUNKS // 2
HALF_SEQ = SEQ // 2
HALF_HID = HID // 2


def _sc_body(hid_hbm, pids_hbm, out_hbm, pids_v, row_v, sem0, sem1, semg):
    nc = 2
    wid = lax.axis_index("s") * nc + lax.axis_index("c")

    @pl.when(wid < 2 * BATCH)
    def _():
        b = wid % BATCH
        half = wid // BATCH
        h0 = half * HALF_HID

        cpg = pltpu.async_copy(
            hid_hbm.at[b, pl.ds(SEQ - 1, 1), pl.ds(h0, HALF_HID)],
            row_v,
            semg,
        )

        cp0 = pltpu.async_copy(
            pids_hbm.at[b, pl.ds(0, HALF_SEQ)], pids_v.at[pl.ds(0, HALF_SEQ)],
            sem0,
        )
        cp1 = pltpu.async_copy(
            pids_hbm.at[b, pl.ds(HALF_SEQ, HALF_SEQ)],
            pids_v.at[pl.ds(HALF_SEQ, HALF_SEQ)],
            sem1,
        )

        init_max = jnp.full((LANES,), jnp.int32(-2147483648), jnp.int32)
        init_chunk = jnp.zeros((LANES,), jnp.int32)

        def body(i, carry):
            cur_max, cur_chunk = carry
            for u in range(UNROLL):
                c = i * UNROLL + u
                v = pids_v[pl.ds(c * LANES, LANES)]
                take = v > cur_max
                cur_max = jnp.where(take, v, cur_max)
                cur_chunk = jnp.where(take, c, cur_chunk)
            return (cur_max, cur_chunk)

        cp0.wait()
        carry = lax.fori_loop(
            0, HALF_CHUNKS // UNROLL, body, (init_max, init_chunk)
        )
        cp1.wait()
        cur_max, cur_chunk = lax.fori_loop(
            HALF_CHUNKS // UNROLL, CHUNKS // UNROLL, body, carry
        )
        cur_idx = cur_chunk * LANES + lax.iota(jnp.int32, LANES)

        best_val = cur_max[0]
        best_idx = cur_idx[0]
        for j in range(1, LANES):
            v = cur_max[j]
            i = cur_idx[j]
            take = (v > best_val) | ((v == best_val) & (i < best_idx))
            best_val = jnp.where(take, v, best_val)
            best_idx = jnp.where(take, i, best_idx)
        idx = best_idx

        cpg.wait()
        pltpu.sync_copy(row_v, out_hbm.at[b, pl.ds(0, 1), pl.ds(h0, HALF_HID)])

        @pl.when(idx != SEQ - 1)
        def _():
            pltpu.sync_copy(
                hid_hbm.at[b, pl.ds(idx, 1), pl.ds(h0, HALF_HID)], row_v
            )
            pltpu.sync_copy(
                row_v, out_hbm.at[b, pl.ds(0, 1), pl.ds(h0, HALF_HID)]
            )


@jax.jit
def _sc_kernel(hidden_states, position_ids):
    return pl.kernel(
        _sc_body,
        mesh=plsc.VectorSubcoreMesh(core_axis_name="c", subcore_axis_name="s"),
        out_type=jax.ShapeDtypeStruct((BATCH, 1, HID), jnp.float32),
        scratch_types=[
            pltpu.VMEM((SEQ,), jnp.int32),
            pltpu.VMEM((1, HALF_HID), jnp.float32),
            pltpu.SemaphoreType.DMA,
            pltpu.SemaphoreType.DMA,
            pltpu.SemaphoreType.DMA,
        ],
    )(hidden_states, position_ids)


def kernel(hidden_states, position_ids):
    return _sc_kernel(hidden_states, position_ids)


# speedup vs baseline: 1.0746x; 1.0201x over previous
import functools

import jax
import jax.numpy as jnp
from jax import lax
from jax.experimental import pallas as pl
from jax.experimental.pallas import tpu as pltpu
from jax.experimental.pallas import tpu_sc as plsc

BATCH = 4
SEQ = 8192
HID = 2048
LANES = 16
UNROLL = 8
CHUNKS = SEQ // LANES
HALF_CHUNKS = CHUNKS // 2
HALF_SEQ = SEQ // 2
HALF_HID = HID // 2


def _sc_body(hid_hbm, pids_hbm, out_hbm, pids_v, row_v, sem0, sem1, semg):
    nc = 2
    wid = lax.axis_index("s") * nc + lax.axis_index("c")

    @pl.when(wid < 2 * BATCH)
    def _():
        b = wid % BATCH
        half = wid // BATCH
        h0 = half * HALF_HID

        cpg = pltpu.async_copy(
            hid_hbm.at[b, pl.ds(SEQ - 1, 1), pl.ds(h0, HALF_HID)],
            row_v,
            semg,
        )

        cp0 = pltpu.async_copy(
            pids_hbm.at[b, pl.ds(0, HALF_SEQ)], pids_v.at[pl.ds(0, HALF_SEQ)],
            sem0,
        )
        cp1 = pltpu.async_copy(
            pids_hbm.at[b, pl.ds(HALF_SEQ, HALF_SEQ)],
            pids_v.at[pl.ds(HALF_SEQ, HALF_SEQ)],
            sem1,
        )

        cpg.wait()
        cpo = pltpu.async_copy(
            row_v, out_hbm.at[b, pl.ds(0, 1), pl.ds(h0, HALF_HID)], semg
        )

        init_max = jnp.full((LANES,), jnp.int32(-2147483648), jnp.int32)
        init_chunk = jnp.zeros((LANES,), jnp.int32)

        def body(i, carry):
            cur_max, cur_chunk = carry
            for u in range(UNROLL):
                c = i * UNROLL + u
                v = pids_v[pl.ds(c * LANES, LANES)]
                take = v > cur_max
                cur_max = jnp.where(take, v, cur_max)
                cur_chunk = jnp.where(take, c, cur_chunk)
            return (cur_max, cur_chunk)

        cp0.wait()
        carry = lax.fori_loop(
            0, HALF_CHUNKS // UNROLL, body, (init_max, init_chunk)
        )
        cp1.wait()
        cur_max, cur_chunk = lax.fori_loop(
            HALF_CHUNKS // UNROLL, CHUNKS // UNROLL, body, carry
        )
        cur_idx = cur_chunk * LANES + lax.iota(jnp.int32, LANES)

        best_val = cur_max[0]
        best_idx = cur_idx[0]
        for j in range(1, LANES):
            v = cur_max[j]
            i = cur_idx[j]
            take = (v > best_val) | ((v == best_val) & (i < best_idx))
            best_val = jnp.where(take, v, best_val)
            best_idx = jnp.where(take, i, best_idx)
        idx = best_idx

        cpo.wait()

        @pl.when(idx != SEQ - 1)
        def _():
            pltpu.sync_copy(
                hid_hbm.at[b, pl.ds(idx, 1), pl.ds(h0, HALF_HID)], row_v
            )
            pltpu.sync_copy(
                row_v, out_hbm.at[b, pl.ds(0, 1), pl.ds(h0, HALF_HID)]
            )


@jax.jit
def _sc_kernel(hidden_states, position_ids):
    return pl.kernel(
        _sc_body,
        mesh=plsc.VectorSubcoreMesh(core_axis_name="c", subcore_axis_name="s"),
        out_type=jax.ShapeDtypeStruct((BATCH, 1, HID), jnp.float32),
        scratch_types=[
            pltpu.VMEM((SEQ,), jnp.int32),
            pltpu.VMEM((1, HALF_HID), jnp.float32),
            pltpu.SemaphoreType.DMA,
            pltpu.SemaphoreType.DMA,
            pltpu.SemaphoreType.DMA,
        ],
    )(hidden_states, position_ids)


def kernel(hidden_states, position_ids):
    return _sc_kernel(hidden_states, position_ids)
